# baseline (speedup 1.0000x reference)
import functools

import jax
import jax.numpy as jnp
from jax import lax
from jax.experimental import pallas as pl
from jax.experimental.pallas import tpu as pltpu
from jax.experimental.pallas import tpu_sc as plsc

_VOCAB = 100000
_D = 1024
_B = 4
_S = 2048
_EPS = 1e-07

_NC = 2
_NS = 16
_NW = _NC * _NS
_SPAN = _S // _NW
_HALF = _SPAN // 2
_K = 32
_NCHUNK = 8
_NV = _D // 16


def _rsqrt_newton(v):
    i = lax.bitcast_convert_type(v, jnp.int32)
    i = jnp.int32(0x5F3759DF) - lax.shift_right_logical(i, jnp.full((16,), 1, jnp.int32))
    r = lax.bitcast_convert_type(i, jnp.float32)
    for _ in range(4):
        r = r * (1.5 - 0.5 * v * r * r)
    return r


def _sc_body(ids_ref, mask_ref, wemb_ref, pos_ref, gamma_ref, beta_ref,
             out_ref, idx_buf, mask_buf, pos_buf, wbuf0, wbuf1,
             gamma_buf, beta_buf, scr, sem0, sem1):
    wid = lax.axis_index("s") * _NC + lax.axis_index("c")
    s0 = wid * _SPAN

    handles = [
        pltpu.async_copy(gamma_ref, gamma_buf, sem0),
        pltpu.async_copy(beta_ref, beta_buf, sem0),
    ]
    for c in range(_NCHUNK):
        h, b = c // 4, c % 4
        base = b * _S + s0 + _HALF * h
        handles.append(pltpu.async_copy(ids_ref.at[pl.ds(base, _K)], idx_buf.at[c], sem0))
        handles.append(pltpu.async_copy(mask_ref.at[pl.ds(base, _K)], mask_buf.at[c], sem0))
    for hd in handles:
        hd.wait()

    def fire_gather(c, buf, sem):
        return pltpu.async_copy(wemb_ref.at[idx_buf.at[c]], buf, sem)

    lanes = jnp.arange(16, dtype=jnp.int32)

    def compute_chunk(buf, c):
        for tg in range(_K // 16):
            rows = lanes + (tg * 16)

            def p1(k, acc):
                s1, s2 = acc
                kk = jnp.full((16,), k, jnp.int32)
                w = plsc.load_gather(buf, [rows, kk])
                p = plsc.load_gather(pos_buf, [rows, kk])
                x = w + p
                plsc.store_scatter(buf, [rows, kk], x)
                return (s1 + x, s2 + x * x)
            z16 = jnp.zeros((16,), jnp.float32)
            s1, s2 = lax.fori_loop(0, _D, p1, (z16, z16))
            mean = s1 * (1.0 / _D)
            var = s2 * (1.0 / _D) - mean * mean
            rstd = _rsqrt_newton(var + _EPS)
            m = mask_buf[c, pl.ds(tg * 16, 16)]
            scr[0, :] = mean
            scr[1, :] = rstd * m
            scr[2, :] = m

            def p2(t2, _):
                meanv = plsc.load_gather(
                    scr, [jnp.full((16,), 0, jnp.int32), jnp.full((16,), t2, jnp.int32)])
                av = plsc.load_gather(
                    scr, [jnp.full((16,), 1, jnp.int32), jnp.full((16,), t2, jnp.int32)])
                mv = plsc.load_gather(
                    scr, [jnp.full((16,), 2, jnp.int32), jnp.full((16,), t2, jnp.int32)])
                t = t2 + tg * 16

                def p2k(k, _2):
                    x = buf[t, pl.ds(k * 16, 16)]
                    g = gamma_buf[pl.ds(k * 16, 16)]
                    bb = beta_buf[pl.ds(k * 16, 16)]
                    buf[t, pl.ds(k * 16, 16)] = (x - meanv) * av * g + bb * mv
                    return 0
                lax.fori_loop(0, _NV, p2k, 0)
                return 0
            lax.fori_loop(0, 16, p2, 0)

    pending = fire_gather(0, wbuf0, sem0)
    for c in range(_NCHUNK):
        h, b = c // 4, c % 4
        buf = wbuf0 if c % 2 == 0 else wbuf1
        if c % 4 == 0:
            pltpu.sync_copy(pos_ref.at[pl.ds(s0 + _HALF * h, _HALF)], pos_buf)
        pending.wait()
        if c + 1 < _NCHUNK:
            nbuf = wbuf1 if c % 2 == 0 else wbuf0
            nsem = sem1 if c % 2 == 0 else sem0
            pending = fire_gather(c + 1, nbuf, nsem)
        compute_chunk(buf, c)
        out_base = b * _S + s0 + _HALF * h
        pltpu.sync_copy(buf, out_ref.at[pl.ds(out_base, _K)])


@jax.jit
def _sc_extract(ids_flat, mask_flat, wemb, pos, gamma, beta):
    mesh = plsc.VectorSubcoreMesh(core_axis_name="c", subcore_axis_name="s")
    run = functools.partial(
        pl.kernel,
        mesh=mesh,
        compiler_params=pltpu.CompilerParams(needs_layout_passes=False),
        out_type=jax.ShapeDtypeStruct((_B * _S, _D), jnp.float32),
        scratch_types=[
            pltpu.VMEM((_NCHUNK, _K), jnp.int32),
            pltpu.VMEM((_NCHUNK, _K), jnp.float32),
            pltpu.VMEM((_HALF, _D), jnp.float32),
            pltpu.VMEM((_K, _D), jnp.float32),
            pltpu.VMEM((_K, _D), jnp.float32),
            pltpu.VMEM((_D,), jnp.float32),
            pltpu.VMEM((_D,), jnp.float32),
            pltpu.VMEM((3, 16), jnp.float32),
            pltpu.SemaphoreType.DMA,
            pltpu.SemaphoreType.DMA,
        ],
    )(_sc_body)
    return run(ids_flat, mask_flat, wemb, pos, gamma, beta)


def kernel(input_ids, mask, word_embeddings, position_embeddings, ln_gamma, ln_beta):
    ids_flat = input_ids.reshape(-1).astype(jnp.int32)
    mask_flat = mask.reshape(-1).astype(jnp.float32)
    out = _sc_extract(ids_flat, mask_flat, word_embeddings,
                      position_embeddings, ln_gamma, ln_beta)
    return out.reshape(_B, _S, _D)


# speedup vs baseline: 2.8464x; 2.8464x over previous
import functools

import jax
import jax.numpy as jnp
from jax import lax
from jax.experimental import pallas as pl
from jax.experimental.pallas import tpu as pltpu
from jax.experimental.pallas import tpu_sc as plsc

_VOCAB = 100000
_D = 1024
_B = 4
_S = 2048
_EPS = 1e-07

_NC = 2
_NS = 16
_NW = _NC * _NS
_SPAN = _S // _NW
_HALF = _SPAN // 2
_K = 32
_NCHUNK = 8
_NV = _D // 16


def _rsqrt_newton(v):
    i = lax.bitcast_convert_type(v, jnp.int32)
    i = jnp.int32(0x5F3759DF) - lax.shift_right_logical(i, jnp.full((16,), 1, jnp.int32))
    r = lax.bitcast_convert_type(i, jnp.float32)
    for _ in range(4):
        r = r * (1.5 - 0.5 * v * r * r)
    return r


def _sc_body(ids_ref, mask_ref, wemb_ref, pos_ref, gb_ref,
             out_ref, idx_buf, mask_buf, pos_buf, wbuf0, wbuf1,
             gb_buf, sem0, sem1):
    wid = lax.axis_index("s") * _NC + lax.axis_index("c")
    s0 = wid * _SPAN

    handles = [
        pltpu.async_copy(gb_ref, gb_buf, sem0),
    ]
    for c in range(_NCHUNK):
        h, b = c // 4, c % 4
        base = b * _S + s0 + _HALF * h
        handles.append(pltpu.async_copy(ids_ref.at[pl.ds(base, _K)], idx_buf.at[c], sem0))
        handles.append(pltpu.async_copy(mask_ref.at[pl.ds(base, _K)], mask_buf.at[c], sem0))
    for hd in handles:
        hd.wait()

    def fire_gather(c, buf, sem):
        return pltpu.async_copy(wemb_ref.at[idx_buf.at[c]], buf, sem)

    def compute_chunk(buf, c):
        def token_body(t, carry):
            def p1(k, acc):
                s1, s2 = acc
                w = buf[t, pl.ds(k * 16, 16)]
                p = pos_buf[t, pl.ds(k * 16, 16)]
                x = w + p
                buf[t, pl.ds(k * 16, 16)] = x
                return (s1 + x, s2 + x * x)
            z16 = jnp.zeros((16,), jnp.float32)
            s1, s2 = lax.fori_loop(0, _NV, p1, (z16, z16), unroll=8)
            tot = jnp.sum(s1)
            tot2 = jnp.sum(s2)
            mean = tot * (1.0 / _D)
            var = tot2 * (1.0 / _D) - mean * mean
            rstd = _rsqrt_newton(jnp.full((16,), var + _EPS, jnp.float32))
            mvec = plsc.load_gather(
                mask_buf,
                [jnp.full((16,), c, jnp.int32), jnp.full((16,), t, jnp.int32)])
            meanv = jnp.full((16,), mean, jnp.float32)
            av = rstd * mvec
            shift16 = jnp.full((16,), 16, jnp.uint32)
            himask = jnp.uint32(0xFFFF0000)

            def p2(k, _):
                x = buf[t, pl.ds(k * 16, 16)]
                gbu = gb_buf[pl.ds(k * 16, 16)]
                g = lax.bitcast_convert_type(lax.shift_left(gbu, shift16), jnp.float32)
                bb = lax.bitcast_convert_type(gbu & himask, jnp.float32)
                buf[t, pl.ds(k * 16, 16)] = (x - meanv) * av * g + bb * mvec
                return 0
            lax.fori_loop(0, _NV, p2, 0, unroll=8)
            return carry
        lax.fori_loop(0, _K, token_body, 0)

    pending = fire_gather(0, wbuf0, sem0)
    for c in range(_NCHUNK):
        h, b = c // 4, c % 4
        buf = wbuf0 if c % 2 == 0 else wbuf1
        if c % 4 == 0:
            pltpu.sync_copy(pos_ref.at[pl.ds(s0 + _HALF * h, _HALF)], pos_buf)
        pending.wait()
        if c + 1 < _NCHUNK:
            nbuf = wbuf1 if c % 2 == 0 else wbuf0
            nsem = sem1 if c % 2 == 0 else sem0
            pending = fire_gather(c + 1, nbuf, nsem)
        compute_chunk(buf, c)
        out_base = b * _S + s0 + _HALF * h
        pltpu.sync_copy(buf, out_ref.at[pl.ds(out_base, _K)])


@jax.jit
def _sc_extract(ids_flat, mask_flat, wemb, pos, gb):
    mesh = plsc.VectorSubcoreMesh(core_axis_name="c", subcore_axis_name="s")
    run = functools.partial(
        pl.kernel,
        mesh=mesh,
        compiler_params=pltpu.CompilerParams(needs_layout_passes=False),
        out_type=jax.ShapeDtypeStruct((_B * _S, _D), jnp.float32),
        scratch_types=[
            pltpu.VMEM((_NCHUNK, _K), jnp.int32),
            pltpu.VMEM((_NCHUNK, _K), jnp.float32),
            pltpu.VMEM((_HALF, _D), jnp.float32),
            pltpu.VMEM((_K, _D), jnp.float32),
            pltpu.VMEM((_K, _D), jnp.float32),
            pltpu.VMEM((_D,), jnp.uint32),
            pltpu.SemaphoreType.DMA,
            pltpu.SemaphoreType.DMA,
        ],
    )(_sc_body)
    return run(ids_flat, mask_flat, wemb, pos, gb)


def kernel(input_ids, mask, word_embeddings, position_embeddings, ln_gamma, ln_beta):
    ids_flat = input_ids.reshape(-1).astype(jnp.int32)
    mask_flat = mask.reshape(-1).astype(jnp.float32)
    g16 = lax.bitcast_convert_type(ln_gamma.astype(jnp.bfloat16), jnp.uint16)
    b16 = lax.bitcast_convert_type(ln_beta.astype(jnp.bfloat16), jnp.uint16)
    gb = (b16.astype(jnp.uint32) << 16) | g16.astype(jnp.uint32)
    out = _sc_extract(ids_flat, mask_flat, word_embeddings,
                      position_embeddings, gb)
    return out.reshape(_B, _S, _D)


# speedup vs baseline: 7.8963x; 2.7741x over previous
import functools

import jax
import jax.numpy as jnp
from jax import lax
from jax.experimental import pallas as pl
from jax.experimental.pallas import tpu as pltpu
from jax.experimental.pallas import tpu_sc as plsc

_VOCAB = 100000
_D = 1024
_B = 4
_S = 2048
_N = _B * _S
_EPS = 1e-07

_NC = 2
_NS = 16
_NW = _NC * _NS
_TPW = _N // _NW
_K = 32
_NCHUNK = _TPW // _K
_NBUF = 3

_R = 512


def _sc_gather_body(ids_ref, wemb_ref, out_ref, idx_buf,
                    b0, b1, b2, gs0, gs1, gs2, ss0, ss1, ss2):
    bufs = (b0, b1, b2)
    gsems = (gs0, gs1, gs2)
    ssems = (ss0, ss1, ss2)
    wid = lax.axis_index("s") * _NC + lax.axis_index("c")
    t0 = wid * _TPW

    idx_handles = [
        pltpu.async_copy(ids_ref.at[pl.ds(t0 + c * _K, _K)], idx_buf.at[c], gs0)
        for c in range(_NCHUNK)
    ]
    for hd in idx_handles:
        hd.wait()

    def fire_gather(c):
        i = c % _NBUF
        return pltpu.async_copy(wemb_ref.at[idx_buf.at[c]], bufs[i], gsems[i])

    def fire_store(c):
        i = c % _NBUF
        return pltpu.async_copy(bufs[i], out_ref.at[pl.ds(t0 + c * _K, _K)], ssems[i])

    gh = {0: fire_gather(0), 1: fire_gather(1)}
    sh = {}
    for c in range(_NCHUNK):
        gh[c].wait()
        sh[c] = fire_store(c)
        nxt = c + 2
        if nxt < _NCHUNK:
            prev = nxt - _NBUF
            if prev >= 0:
                sh[prev].wait()
            gh[nxt] = fire_gather(nxt)
    sh[_NCHUNK - 2].wait()
    sh[_NCHUNK - 1].wait()


@jax.jit
def _sc_gather(ids_flat, wemb):
    mesh = plsc.VectorSubcoreMesh(core_axis_name="c", subcore_axis_name="s")
    run = functools.partial(
        pl.kernel,
        mesh=mesh,
        compiler_params=pltpu.CompilerParams(needs_layout_passes=False),
        out_type=jax.ShapeDtypeStruct((_N, _D), jnp.float32),
        scratch_types=[
            pltpu.VMEM((_NCHUNK, _K), jnp.int32),
            pltpu.VMEM((_K, _D), jnp.float32),
            pltpu.VMEM((_K, _D), jnp.float32),
            pltpu.VMEM((_K, _D), jnp.float32),
            pltpu.SemaphoreType.DMA,
            pltpu.SemaphoreType.DMA,
            pltpu.SemaphoreType.DMA,
            pltpu.SemaphoreType.DMA,
            pltpu.SemaphoreType.DMA,
            pltpu.SemaphoreType.DMA,
        ],
    )(_sc_gather_body)
    return run(ids_flat, wemb)


def _tc_ln_body(x_ref, p_ref, m_ref, g_ref, b_ref, o_ref):
    x = x_ref[...] + p_ref[...]
    s1 = jnp.sum(x, axis=1, keepdims=True)
    s2 = jnp.sum(x * x, axis=1, keepdims=True)
    mean = s1 * (1.0 / _D)
    var = s2 * (1.0 / _D) - mean * mean
    y = (x - mean) * lax.rsqrt(var + _EPS)
    o_ref[...] = (g_ref[...] * y + b_ref[...]) * m_ref[...]


@jax.jit
def _tc_ln(gathered, pos, mask2d, gamma2d, beta2d):
    grid = (_N // _R,)
    return pl.pallas_call(
        _tc_ln_body,
        grid=grid,
        in_specs=[
            pl.BlockSpec((_R, _D), lambda i: (i, 0)),
            pl.BlockSpec((_R, _D), lambda i: (i % (_S // _R), 0)),
            pl.BlockSpec((_R, 1), lambda i: (i, 0)),
            pl.BlockSpec((1, _D), lambda i: (0, 0)),
            pl.BlockSpec((1, _D), lambda i: (0, 0)),
        ],
        out_specs=pl.BlockSpec((_R, _D), lambda i: (i, 0)),
        out_shape=jax.ShapeDtypeStruct((_N, _D), jnp.float32),
    )(gathered, pos, mask2d, gamma2d, beta2d)


def kernel(input_ids, mask, word_embeddings, position_embeddings, ln_gamma, ln_beta):
    ids_flat = input_ids.reshape(-1).astype(jnp.int32)
    gathered = _sc_gather(ids_flat, word_embeddings)
    out = _tc_ln(
        gathered,
        position_embeddings,
        mask.reshape(_N, 1).astype(jnp.float32),
        ln_gamma.reshape(1, _D),
        ln_beta.reshape(1, _D),
    )
    return out.reshape(_B, _S, _D)


# speedup vs baseline: 8.9240x; 1.1301x over previous
import functools

import jax
import jax.numpy as jnp
from jax import lax
from jax.experimental import pallas as pl
from jax.experimental.pallas import tpu as pltpu
from jax.experimental.pallas import tpu_sc as plsc

_VOCAB = 100000
_D = 1024
_B = 4
_S = 2048
_N = _B * _S
_EPS = 1e-07

_NC = 2
_NS = 16
_NW = _NC * _NS
_TPW = _N // _NW
_K = 32
_NCHUNK = _TPW // _K
_NBUF = 3

_R = 512


def _sc_gather_body(ids_ref, wemb_ref, out_ref, idx_buf,
                    b0, b1, b2, gs0, gs1, gs2, ss0, ss1, ss2):
    bufs = (b0, b1, b2)
    gsems = (gs0, gs1, gs2)
    ssems = (ss0, ss1, ss2)
    wid = lax.axis_index("s") * _NC + lax.axis_index("c")
    t0 = wid * _TPW

    idx_handles = [
        pltpu.async_copy(ids_ref.at[pl.ds(t0 + c * _K, _K)], idx_buf.at[c], gs0)
        for c in range(_NCHUNK)
    ]
    for hd in idx_handles:
        hd.wait()

    def fire_gather(c):
        i = c % _NBUF
        return pltpu.async_copy(wemb_ref.at[idx_buf.at[c]], bufs[i], gsems[i])

    def fire_store(c):
        i = c % _NBUF
        return pltpu.async_copy(bufs[i], out_ref.at[pl.ds(t0 + c * _K, _K)], ssems[i])

    gh = {0: fire_gather(0), 1: fire_gather(1)}
    sh = {}
    for c in range(_NCHUNK):
        gh[c].wait()
        sh[c] = fire_store(c)
        nxt = c + 2
        if nxt < _NCHUNK:
            prev = nxt - _NBUF
            if prev >= 0:
                sh[prev].wait()
            gh[nxt] = fire_gather(nxt)
    sh[_NCHUNK - 2].wait()
    sh[_NCHUNK - 1].wait()


@jax.jit
def _sc_gather(ids_flat, wemb):
    mesh = plsc.VectorSubcoreMesh(core_axis_name="c", subcore_axis_name="s")
    run = functools.partial(
        pl.kernel,
        mesh=mesh,
        compiler_params=pltpu.CompilerParams(needs_layout_passes=False),
        out_type=jax.ShapeDtypeStruct((_N, _D), jnp.float32),
        scratch_types=[
            pltpu.VMEM((_NCHUNK, _K), jnp.int32),
            pltpu.VMEM((_K, _D), jnp.float32),
            pltpu.VMEM((_K, _D), jnp.float32),
            pltpu.VMEM((_K, _D), jnp.float32),
            pltpu.SemaphoreType.DMA,
            pltpu.SemaphoreType.DMA,
            pltpu.SemaphoreType.DMA,
            pltpu.SemaphoreType.DMA,
            pltpu.SemaphoreType.DMA,
            pltpu.SemaphoreType.DMA,
        ],
    )(_sc_gather_body)
    return run(ids_flat, wemb)


def _tc_ln_body(x_ref, p_ref, m_ref, g_ref, b_ref, o_ref):
    x = x_ref[...] + p_ref[...][None, :, :]
    s1 = jnp.sum(x, axis=2, keepdims=True)
    s2 = jnp.sum(x * x, axis=2, keepdims=True)
    mean = s1 * (1.0 / _D)
    var = s2 * (1.0 / _D) - mean * mean
    y = (x - mean) * lax.rsqrt(var + _EPS)
    o_ref[...] = (g_ref[...][None] * y + b_ref[...][None]) * m_ref[...]


@jax.jit
def _tc_ln(gathered3d, pos, mask3d, gamma2d, beta2d):
    grid = (_S // _R,)
    return pl.pallas_call(
        _tc_ln_body,
        grid=grid,
        in_specs=[
            pl.BlockSpec((_B, _R, _D), lambda i: (0, i, 0)),
            pl.BlockSpec((_R, _D), lambda i: (i, 0)),
            pl.BlockSpec((_B, _R, 1), lambda i: (0, i, 0)),
            pl.BlockSpec((1, _D), lambda i: (0, 0)),
            pl.BlockSpec((1, _D), lambda i: (0, 0)),
        ],
        out_specs=pl.BlockSpec((_B, _R, _D), lambda i: (0, i, 0)),
        out_shape=jax.ShapeDtypeStruct((_B, _S, _D), jnp.float32),
    )(gathered3d, pos, mask3d, gamma2d, beta2d)


def kernel(input_ids, mask, word_embeddings, position_embeddings, ln_gamma, ln_beta):
    ids_flat = input_ids.reshape(-1).astype(jnp.int32)
    gathered = _sc_gather(ids_flat, word_embeddings)
    return _tc_ln(
        gathered.reshape(_B, _S, _D),
        position_embeddings,
        mask.reshape(_B, _S, 1).astype(jnp.float32),
        ln_gamma.reshape(1, _D),
        ln_beta.reshape(1, _D),
    )
